# Initial kernel scaffold; baseline (speedup 1.0000x reference)
import jax
import jax.numpy as jnp
from jax.experimental import pallas as pl


def kernel(ids, embedding_users, embedding_items, bias_users, bias_items, global_bias):
    raise NotImplementedError("write your pallas kernel here")


# speedup vs baseline: 23.7897x; 23.7897x over previous
import functools

import jax
import jax.numpy as jnp
from jax import lax
from jax.experimental import pallas as pl
from jax.experimental.pallas import tpu as pltpu
from jax.experimental.pallas import tpu_sc as plsc

EMBED_DIM = 32
NUM_CORES = 2
NUM_SUBCORES = 16
NUM_WORKERS = NUM_CORES * NUM_SUBCORES
LANES = 16
IDX_CHUNK = 128


@functools.lru_cache(maxsize=None)
def _build_mf_kernel(batch: int):
    assert batch % (NUM_WORKERS * LANES) == 0
    b_per_w = batch // NUM_WORKERS
    n_chunks = b_per_w // IDX_CHUNK
    n_groups = b_per_w // LANES
    mesh = plsc.VectorSubcoreMesh(
        core_axis_name="c", subcore_axis_name="s", num_cores=NUM_CORES
    )

    @functools.partial(
        pl.kernel,
        mesh=mesh,
        compiler_params=pltpu.CompilerParams(
            needs_layout_passes=False, use_tc_tiling_on_sc=False
        ),
        out_type=jax.ShapeDtypeStruct((batch,), jnp.float32),
        scratch_types=[
            pltpu.VMEM((n_chunks, IDX_CHUNK), jnp.int32),
            pltpu.VMEM((n_chunks, IDX_CHUNK), jnp.int32),
            pltpu.VMEM((b_per_w, EMBED_DIM), jnp.float32),
            pltpu.VMEM((b_per_w, EMBED_DIM), jnp.float32),
            pltpu.VMEM((b_per_w,), jnp.float32),
            pltpu.VMEM((b_per_w,), jnp.float32),
            pltpu.VMEM((LANES,), jnp.float32),
            pltpu.VMEM((b_per_w,), jnp.float32),
            pltpu.SemaphoreType.DMA,
        ],
    )
    def mf_kernel(
        uid_hbm, iid_hbm, utab_hbm, itab_hbm, ubias_hbm, ibias_hbm, gb_hbm,
        out_hbm,
        uid_v, iid_v, urows_v, irows_v, ubias_v, ibias_v, gb_v, out_v,
        sem,
    ):
        wid = lax.axis_index("s") * NUM_CORES + lax.axis_index("c")
        base = wid * b_per_w

        for j in range(n_chunks):
            pltpu.sync_copy(
                uid_hbm.at[pl.ds(base + j * IDX_CHUNK, IDX_CHUNK)],
                uid_v.at[j],
            )
            pltpu.sync_copy(
                iid_hbm.at[pl.ds(base + j * IDX_CHUNK, IDX_CHUNK)],
                iid_v.at[j],
            )
        pltpu.sync_copy(gb_hbm, gb_v)

        copies = []
        for j in range(n_chunks):
            off = j * IDX_CHUNK
            copies.append(pltpu.async_copy(
                utab_hbm.at[uid_v.at[j]],
                urows_v.at[pl.ds(off, IDX_CHUNK), :], sem))
            copies.append(pltpu.async_copy(
                itab_hbm.at[iid_v.at[j]],
                irows_v.at[pl.ds(off, IDX_CHUNK), :], sem))
            copies.append(pltpu.async_copy(
                ubias_hbm.at[uid_v.at[j]],
                ubias_v.at[pl.ds(off, IDX_CHUNK)], sem))
            copies.append(pltpu.async_copy(
                ibias_hbm.at[iid_v.at[j]],
                ibias_v.at[pl.ds(off, IDX_CHUNK)], sem))
        for cp in copies:
            cp.wait()

        gb = gb_v[...]

        def init_body(g, carry):
            off = g * LANES
            out_v[pl.ds(off, LANES)] = (
                ubias_v[pl.ds(off, LANES)] + ibias_v[pl.ds(off, LANES)] + gb
            )
            return carry

        lax.fori_loop(0, n_groups, init_body, 0)

        ROW_UNROLL = 8

        def dot_body(rr, carry):
            r0 = rr * ROW_UNROLL
            for u in range(ROW_UNROLL):
                r = r0 + u
                prod = (
                    urows_v[r, pl.ds(0, LANES)] * irows_v[r, pl.ds(0, LANES)]
                    + urows_v[r, pl.ds(LANES, LANES)]
                    * irows_v[r, pl.ds(LANES, LANES)]
                )
                ridx = jnp.full((LANES,), r, jnp.int32)
                plsc.addupdate_scatter(out_v, [ridx], prod)
            return carry

        lax.fori_loop(0, b_per_w // ROW_UNROLL, dot_body, 0)
        pltpu.sync_copy(out_v, out_hbm.at[pl.ds(base, b_per_w)])

    return mf_kernel


def kernel(ids, embedding_users, embedding_items, bias_users, bias_items,
           global_bias):
    batch = ids.shape[0]
    uid = ids[:, 0].astype(jnp.int32)
    iid = ids[:, 1].astype(jnp.int32)
    utab = embedding_users.reshape(-1, EMBED_DIM)
    itab = embedding_items.reshape(-1, EMBED_DIM)
    gb = jnp.broadcast_to(global_bias.astype(jnp.float32), (LANES,))
    return _build_mf_kernel(batch)(
        uid, iid, utab, itab, bias_users, bias_items, gb
    )


# speedup vs baseline: 29.6135x; 1.2448x over previous
import functools

import jax
import jax.numpy as jnp
from jax import lax
from jax.experimental import pallas as pl
from jax.experimental.pallas import tpu as pltpu
from jax.experimental.pallas import tpu_sc as plsc

EMBED_DIM = 32
NUM_CORES = 2
NUM_SUBCORES = 16
NUM_WORKERS = NUM_CORES * NUM_SUBCORES
LANES = 16
IDX_CHUNK = 128
ROW_UNROLL = 8


@functools.lru_cache(maxsize=None)
def _build_mf_kernel(batch: int):
    assert batch % (NUM_WORKERS * LANES) == 0
    b_per_w = batch // NUM_WORKERS
    n_chunks = b_per_w // IDX_CHUNK
    n_groups = b_per_w // LANES
    mesh = plsc.VectorSubcoreMesh(
        core_axis_name="c", subcore_axis_name="s", num_cores=NUM_CORES
    )

    @functools.partial(
        pl.kernel,
        mesh=mesh,
        compiler_params=pltpu.CompilerParams(
            needs_layout_passes=False, use_tc_tiling_on_sc=False
        ),
        out_type=jax.ShapeDtypeStruct((batch,), jnp.float32),
        scratch_types=[
            pltpu.VMEM((n_chunks, IDX_CHUNK), jnp.int32),
            pltpu.VMEM((n_chunks, IDX_CHUNK), jnp.int32),
            pltpu.VMEM((b_per_w, EMBED_DIM), jnp.float32),
            pltpu.VMEM((b_per_w, EMBED_DIM), jnp.float32),
            pltpu.VMEM((b_per_w,), jnp.float32),
            pltpu.VMEM((b_per_w,), jnp.float32),
            pltpu.VMEM((LANES,), jnp.float32),
            pltpu.VMEM((LANES * b_per_w,), jnp.float32),
            pltpu.VMEM((b_per_w,), jnp.float32),
        ] + [pltpu.SemaphoreType.DMA] * (n_chunks + 1),
    )
    def mf_kernel(
        uid_hbm, iid_hbm, utab_hbm, itab_hbm, ubias_hbm, ibias_hbm, gb_hbm,
        out_hbm,
        uid_v, iid_v, urows_v, irows_v, ubias_v, ibias_v, gb_v, pt_v, out_v,
        *sems,
    ):
        row_sems = sems[:n_chunks]
        bias_sem = sems[n_chunks]
        wid = lax.axis_index("s") * NUM_CORES + lax.axis_index("c")
        base = wid * b_per_w

        pltpu.sync_copy(uid_hbm.at[wid], uid_v)
        pltpu.sync_copy(iid_hbm.at[wid], iid_v)

        row_copies = []
        for c in range(n_chunks):
            off = c * IDX_CHUNK
            row_copies.append((
                pltpu.async_copy(
                    utab_hbm.at[uid_v.at[c]],
                    urows_v.at[pl.ds(off, IDX_CHUNK), :], row_sems[c]),
                pltpu.async_copy(
                    itab_hbm.at[iid_v.at[c]],
                    irows_v.at[pl.ds(off, IDX_CHUNK), :], row_sems[c]),
            ))
        bias_copies = []
        for c in range(n_chunks):
            off = c * IDX_CHUNK
            bias_copies.append(pltpu.async_copy(
                ubias_hbm.at[uid_v.at[c]],
                ubias_v.at[pl.ds(off, IDX_CHUNK)], bias_sem))
            bias_copies.append(pltpu.async_copy(
                ibias_hbm.at[iid_v.at[c]],
                ibias_v.at[pl.ds(off, IDX_CHUNK)], bias_sem))
        pltpu.sync_copy(gb_hbm, gb_v)

        lane_off = lax.iota(jnp.int32, LANES) * b_per_w

        for c in range(n_chunks):
            for cp in row_copies[c]:
                cp.wait()

            def dot_body(rr, carry, c=c):
                r = c * IDX_CHUNK + rr * ROW_UNROLL
                for u in range(ROW_UNROLL):
                    ru = r + u
                    prod = (
                        urows_v[ru, pl.ds(0, LANES)]
                        * irows_v[ru, pl.ds(0, LANES)]
                        + urows_v[ru, pl.ds(LANES, LANES)]
                        * irows_v[ru, pl.ds(LANES, LANES)]
                    )
                    plsc.store_scatter(pt_v, [lane_off + ru], prod)
                return carry

            lax.fori_loop(0, IDX_CHUNK // ROW_UNROLL, dot_body, 0)

        for cp in bias_copies:
            cp.wait()
        gb = gb_v[...]

        def fold_body(g, carry):
            off = g * LANES
            acc = ubias_v[pl.ds(off, LANES)] + ibias_v[pl.ds(off, LANES)] + gb
            for l in range(LANES):
                acc = acc + pt_v[pl.ds(l * b_per_w + off, LANES)]
            out_v[pl.ds(off, LANES)] = acc
            return carry

        lax.fori_loop(0, n_groups, fold_body, 0)
        pltpu.sync_copy(out_v, out_hbm.at[pl.ds(base, b_per_w)])

    return mf_kernel


def kernel(ids, embedding_users, embedding_items, bias_users, bias_items,
           global_bias):
    batch = ids.shape[0]
    uid = ids[:, 0].astype(jnp.int32).reshape(NUM_WORKERS, -1, IDX_CHUNK)
    iid = ids[:, 1].astype(jnp.int32).reshape(NUM_WORKERS, -1, IDX_CHUNK)
    utab = embedding_users.reshape(-1, EMBED_DIM)
    itab = embedding_items.reshape(-1, EMBED_DIM)
    gb = jnp.broadcast_to(global_bias.astype(jnp.float32), (LANES,))
    return _build_mf_kernel(batch)(
        uid, iid, utab, itab, bias_users, bias_items, gb
    )


# speedup vs baseline: 30.1350x; 1.0176x over previous
import functools

import jax
import jax.numpy as jnp
from jax import lax
from jax.experimental import pallas as pl
from jax.experimental.pallas import tpu as pltpu
from jax.experimental.pallas import tpu_sc as plsc

EMBED_DIM = 32
NUM_CORES = 2
NUM_SUBCORES = 16
NUM_WORKERS = NUM_CORES * NUM_SUBCORES
LANES = 16
ROW_UNROLL = 8


@functools.lru_cache(maxsize=None)
def _build_mf_kernel(batch: int):
    assert batch % (NUM_WORKERS * LANES) == 0
    b_per_w = batch // NUM_WORKERS
    n_groups = b_per_w // LANES
    mesh = plsc.VectorSubcoreMesh(
        core_axis_name="c", subcore_axis_name="s", num_cores=NUM_CORES
    )

    @functools.partial(
        pl.kernel,
        mesh=mesh,
        compiler_params=pltpu.CompilerParams(
            needs_layout_passes=False, use_tc_tiling_on_sc=False
        ),
        out_type=jax.ShapeDtypeStruct((batch,), jnp.float32),
        scratch_types=[
            pltpu.VMEM((2, b_per_w), jnp.int32),
            pltpu.VMEM((b_per_w, EMBED_DIM), jnp.float32),
            pltpu.VMEM((b_per_w, EMBED_DIM), jnp.float32),
            pltpu.VMEM((b_per_w,), jnp.float32),
            pltpu.VMEM((b_per_w,), jnp.float32),
            pltpu.VMEM((LANES,), jnp.float32),
            pltpu.VMEM((LANES * b_per_w,), jnp.float32),
            pltpu.VMEM((b_per_w,), jnp.float32),
            pltpu.SemaphoreType.DMA,
        ],
    )
    def mf_kernel(
        ids_hbm, utab_hbm, itab_hbm, ubias_hbm, ibias_hbm, gb_hbm,
        out_hbm,
        idx_v, urows_v, irows_v, ubias_v, ibias_v, gb_v, pt_v, out_v,
        sem,
    ):
        wid = lax.axis_index("s") * NUM_CORES + lax.axis_index("c")
        base = wid * b_per_w

        pltpu.sync_copy(ids_hbm.at[wid], idx_v)

        copies = (
            pltpu.async_copy(utab_hbm.at[idx_v.at[0]], urows_v, sem),
            pltpu.async_copy(itab_hbm.at[idx_v.at[1]], irows_v, sem),
            pltpu.async_copy(ubias_hbm.at[idx_v.at[0]], ubias_v, sem),
            pltpu.async_copy(ibias_hbm.at[idx_v.at[1]], ibias_v, sem),
        )
        pltpu.sync_copy(gb_hbm, gb_v)
        for cp in copies:
            cp.wait()

        lane_off = lax.iota(jnp.int32, LANES) * b_per_w

        def dot_body(rr, carry):
            r = rr * ROW_UNROLL
            for u in range(ROW_UNROLL):
                ru = r + u
                prod = (
                    urows_v[ru, pl.ds(0, LANES)]
                    * irows_v[ru, pl.ds(0, LANES)]
                    + urows_v[ru, pl.ds(LANES, LANES)]
                    * irows_v[ru, pl.ds(LANES, LANES)]
                )
                plsc.store_scatter(pt_v, [lane_off + ru], prod)
            return carry

        lax.fori_loop(0, b_per_w // ROW_UNROLL, dot_body, 0)

        gb = gb_v[...]

        def fold_body(g, carry):
            off = g * LANES
            acc = ubias_v[pl.ds(off, LANES)] + ibias_v[pl.ds(off, LANES)] + gb
            for l in range(LANES):
                acc = acc + pt_v[pl.ds(l * b_per_w + off, LANES)]
            out_v[pl.ds(off, LANES)] = acc
            return carry

        lax.fori_loop(0, n_groups, fold_body, 0)
        pltpu.sync_copy(out_v, out_hbm.at[pl.ds(base, b_per_w)])

    return mf_kernel


def kernel(ids, embedding_users, embedding_items, bias_users, bias_items,
           global_bias):
    batch = ids.shape[0]
    b_per_w = batch // NUM_WORKERS
    idall = (
        ids.astype(jnp.int32).reshape(NUM_WORKERS, b_per_w, 2)
        .transpose(0, 2, 1)
    )
    utab = embedding_users.reshape(-1, EMBED_DIM)
    itab = embedding_items.reshape(-1, EMBED_DIM)
    gb = jnp.broadcast_to(global_bias.astype(jnp.float32), (LANES,))
    return _build_mf_kernel(batch)(
        idall, utab, itab, bias_users, bias_items, gb
    )
